# baseline (speedup 1.0000x reference)
import functools

import jax
import jax.numpy as jnp
from jax import lax
from jax.experimental import pallas as pl
from jax.experimental.pallas import tpu as pltpu
from jax.experimental.pallas import tpu_sc as plsc

NC = 2
NS = 16
LANES = 16
NW = NC * NS

B_EDGE = 32


def _mm_bias_body(x_ref, w_ref, b_ref, o_ref):
    o_ref[...] = (
        jnp.dot(x_ref[...], w_ref[...], preferred_element_type=jnp.float32)
        + b_ref[...]
    )


def _tc_mm_bias(x, w, b, bn):
    n, d = x.shape
    k = w.shape[1]
    return pl.pallas_call(
        _mm_bias_body,
        grid=(n // bn,),
        in_specs=[
            pl.BlockSpec((bn, d), lambda i: (i, 0)),
            pl.BlockSpec((d, k), lambda i: (0, 0)),
            pl.BlockSpec((1, k), lambda i: (0, 0)),
        ],
        out_specs=pl.BlockSpec((bn, k), lambda i: (i, 0)),
        out_shape=jax.ShapeDtypeStruct((n, k), jnp.float32),
    )(x, w, b.reshape(1, k))


def _proj2_body(h_ref, w1_ref, w2_ref, o1_ref, o2_ref):
    h = h_ref[...]
    o1_ref[...] = jnp.dot(h, w1_ref[...], preferred_element_type=jnp.float32)
    o2_ref[...] = jnp.dot(h, w2_ref[...], preferred_element_type=jnp.float32)


def _tc_proj2(h, w1, w2, bn=2000):
    n, d = h.shape
    k = w1.shape[1]
    out = jax.ShapeDtypeStruct((n, k), jnp.float32)
    return pl.pallas_call(
        _proj2_body,
        grid=(n // bn,),
        in_specs=[
            pl.BlockSpec((bn, d), lambda i: (i, 0)),
            pl.BlockSpec((d, k), lambda i: (0, 0)),
            pl.BlockSpec((d, k), lambda i: (0, 0)),
        ],
        out_specs=[pl.BlockSpec((bn, k), lambda i: (i, 0))] * 2,
        out_shape=[out, out],
    )(h, w1, w2)


def _combine_proj4_body(h_ref, a0_ref, a1_ref, w1_ref, w2_ref, w3_ref, w4_ref,
                        hn_ref, o1_ref, o2_ref, o3_ref, o4_ref):
    hn = jnp.maximum(h_ref[...] + a0_ref[...] + a1_ref[...], 0.0)
    hn_ref[...] = hn
    for w_ref, o_ref in ((w1_ref, o1_ref), (w2_ref, o2_ref),
                         (w3_ref, o3_ref), (w4_ref, o4_ref)):
        o_ref[...] = jnp.dot(hn, w_ref[...], preferred_element_type=jnp.float32)


def _tc_combine_proj4(h, a0, a1, w1, w2, w3, w4, bn=1000):
    n, d = h.shape
    k = w1.shape[1]
    big = jax.ShapeDtypeStruct((n, k), jnp.float32)
    bspec = pl.BlockSpec((bn, d), lambda i: (i, 0))
    wspec = pl.BlockSpec((d, k), lambda i: (0, 0))
    ospec = pl.BlockSpec((bn, k), lambda i: (i, 0))
    return pl.pallas_call(
        _combine_proj4_body,
        grid=(n // bn,),
        in_specs=[bspec, bspec, bspec, wspec, wspec, wspec, wspec],
        out_specs=[bspec, ospec, ospec, ospec, ospec],
        out_shape=[jax.ShapeDtypeStruct((n, d), jnp.float32), big, big, big, big],
    )(h, a0, a1, w1, w2, w3, w4)


def _edgec_body(ea_ref, wf_ref, ws_ref, bf_ref, bs_ref, o_ref, *, bn, e_real):
    ea = ea_ref[...]
    cf = jnp.dot(ea, wf_ref[...], preferred_element_type=jnp.float32) + bf_ref[...]
    cs = jnp.dot(ea, ws_ref[...], preferred_element_type=jnp.float32) + bs_ref[...]
    rid = pl.program_id(0) * bn + lax.broadcasted_iota(jnp.int32, (bn, 1), 0)
    cs = jnp.where(rid >= e_real, -1e9, cs)
    o_ref[...] = jnp.concatenate([cf, cs], axis=1)


def _tc_edgec(ea_pad, wf_e, ws_e, bf, bs, e_real, bn=5008):
    n_pad, de = ea_pad.shape
    k = wf_e.shape[1]
    return pl.pallas_call(
        functools.partial(_edgec_body, bn=bn, e_real=e_real),
        grid=(n_pad // bn,),
        in_specs=[
            pl.BlockSpec((bn, de), lambda i: (i, 0)),
            pl.BlockSpec((de, k), lambda i: (0, 0)),
            pl.BlockSpec((de, k), lambda i: (0, 0)),
            pl.BlockSpec((1, k), lambda i: (0, 0)),
            pl.BlockSpec((1, k), lambda i: (0, 0)),
        ],
        out_specs=pl.BlockSpec((bn, 2 * k), lambda i: (i, 0)),
        out_shape=jax.ShapeDtypeStruct((n_pad, 2 * k), jnp.float32),
    )(ea_pad, wf_e, ws_e, bf.reshape(1, k), bs.reshape(1, k))


def _combine_body(h_ref, a0_ref, a1_ref, o_ref):
    o_ref[...] = h_ref[...] + a0_ref[...] + a1_ref[...]


def _tc_combine(h, a0, a1, bn=2000):
    n, d = h.shape
    spec = pl.BlockSpec((bn, d), lambda i: (i, 0))
    return pl.pallas_call(
        _combine_body,
        grid=(n // bn,),
        in_specs=[spec, spec, spec],
        out_specs=spec,
        out_shape=jax.ShapeDtypeStruct((n, d), jnp.float32),
    )(h, a0, a1)


def _gate_softplus16(gf, gs):
    u = jnp.exp(-jnp.abs(gs))
    t = u / (u + 2.0)
    t2 = t * t
    l1p = (2.0 * t) * (1.0 + t2 * (1.0 / 3.0 + t2 * (0.2 + t2 * (1.0 / 7.0))))
    sp = jnp.maximum(gs, 0.0) + l1p
    return sp / (1.0 + jnp.exp(-gf))


def _make_sc_edge(n, d, e):
    base_rows = (n // NS) // 8 * 8
    cz = 208
    ncz = base_rows // cz
    tail = n - NS * base_rows
    tail_row = NS * base_rows
    ew = e // NW
    nblk = ew // B_EDGE
    assert base_rows % cz == 0 and tail % 8 == 0
    assert e % NW == 0 and ew % B_EDGE == 0 and B_EDGE % 8 == 0
    assert nblk % 2 == 0 and nblk >= 4

    mesh = plsc.VectorSubcoreMesh(
        core_axis_name="c", subcore_axis_name="s", num_cores=NC,
        num_subcores=NS)

    @functools.partial(
        pl.kernel,
        mesh=mesh,
        out_type=jax.ShapeDtypeStruct((NC, n, d), jnp.float32),
        scratch_types=[
            pltpu.VMEM((2, B_EDGE), jnp.int32),
            pltpu.VMEM((2, B_EDGE), jnp.int32),
            pltpu.VMEM((B_EDGE,), jnp.int32),
            pltpu.VMEM((B_EDGE,), jnp.int32),
            pltpu.VMEM((B_EDGE, 2 * d), jnp.float32),
            pltpu.VMEM((B_EDGE, 2 * d), jnp.float32),
            pltpu.VMEM((B_EDGE, 2 * d), jnp.float32),
            pltpu.VMEM((B_EDGE, 2 * d), jnp.float32),
            pltpu.VMEM((B_EDGE, 2 * d), jnp.float32),
            pltpu.VMEM((B_EDGE, d), jnp.float32),
            pltpu.VMEM_SHARED((n, d), jnp.float32),
        ] + [pltpu.SemaphoreType.DMA] * 7,
    )
    def sc_edge(ei_hbm, pd_hbm, ps_hbm, c_hbm, zero_hbm, out_hbm,
                dsrc0, dsrc1, sdst0, sdst1, pdv0, pdv1, psv0, psv1, cv,
                mv, acc, si0, si1, spd0, spd1, sps0, sps1, scc):
        ci = lax.axis_index("c")
        si = lax.axis_index("s")
        wid = si * NC + ci
        base_blk = wid * nblk
        row0 = si * base_rows

        bufs = ((dsrc0, sdst0, pdv0, psv0, si0, spd0, sps0),
                (dsrc1, sdst1, pdv1, psv1, si1, spd1, sps1))

        for kk in range(ncz):
            pltpu.sync_copy(zero_hbm, acc.at[pl.ds(row0 + kk * cz, cz), :])

        @pl.when(si == NS - 1)
        def _zero_tail():
            pltpu.sync_copy(zero_hbm.at[pl.ds(0, tail), :],
                            acc.at[pl.ds(tail_row, tail), :])

        plsc.subcore_barrier()

        def idx_fetch(blk, b):
            buf = bufs[b]
            pltpu.async_copy(ei_hbm.at[blk], buf[0], buf[4])

        def idx_wait(b):
            buf = bufs[b]
            pltpu.make_async_copy(ei_hbm.at[0], buf[0], buf[4]).wait()

        def cfetch(blk):
            pltpu.async_copy(c_hbm.at[pl.ds(blk * B_EDGE, B_EDGE), :], cv, scc)

        def gather(blk, jb, b):
            dsrc = bufs[jb][0]
            _, _, pdv, psv, _, spd, sps = bufs[b]
            pltpu.async_copy(pd_hbm.at[dsrc.at[0]], pdv, spd)
            pltpu.async_copy(ps_hbm.at[dsrc.at[1]], psv, sps)

        def consume(b, next_blk, do_fetch, cnext_blk, do_cfetch, unroll):
            dsrc, sdst, pdv, psv, _, spd, sps = bufs[b]
            pltpu.make_async_copy(pd_hbm.at[dsrc.at[0]], pdv, spd).wait()
            pltpu.make_async_copy(ps_hbm.at[dsrc.at[1]], psv, sps).wait()
            pltpu.make_async_copy(c_hbm.at[pl.ds(0, B_EDGE), :], cv, scc).wait()
            for q0 in range(0, B_EDGE - LANES + 1, LANES):
                sdst[pl.ds(q0, LANES)] = dsrc[0, pl.ds(q0, LANES)]
            if B_EDGE % LANES:
                q0 = B_EDGE - LANES
                sdst[pl.ds(q0, LANES)] = dsrc[0, pl.ds(q0, LANES)]
            if do_fetch:
                idx_fetch(next_blk, b)

            @plsc.parallel_loop(0, B_EDGE, 1, unroll=unroll)
            def edge_body(ei):
                for jj in range(d // LANES):
                    o = jj * LANES
                    cf = cv[ei, pl.ds(o, LANES)]
                    cs = cv[ei, pl.ds(d + o, LANES)]
                    gf = pdv[ei, pl.ds(o, LANES)] + psv[ei, pl.ds(o, LANES)] + cf
                    gs = (pdv[ei, pl.ds(d + o, LANES)]
                          + psv[ei, pl.ds(d + o, LANES)] + cs)
                    mv[ei, pl.ds(o, LANES)] = _gate_softplus16(gf, gs)
            pltpu.sync_copy(mv, acc.at[sdst], add=True)
            if do_cfetch:
                cfetch(cnext_blk)

        idx_fetch(base_blk, 0)
        idx_wait(0)
        gather(base_blk, 0, 0)
        cfetch(base_blk)
        idx_fetch(base_blk + 1, 1)

        def pair_body(i, carry):
            k = base_blk + 2 * i
            idx_wait(1)
            gather(k + 1, 1, 1)
            consume(0, k + 2, True, k + 1, True, 2)
            idx_wait(0)
            gather(k + 2, 0, 0)
            consume(1, k + 3, True, k + 2, True, 2)
            return carry

        lax.fori_loop(0, (nblk - 2) // 2, pair_body, 0, unroll=False)
        idx_wait(1)
        gather(base_blk + nblk - 1, 1, 1)
        consume(0, 0, False, base_blk + nblk - 1, True, 1)
        consume(1, 0, False, 0, False, 1)
        plsc.subcore_barrier()

        for kk in range(ncz):
            r = row0 + kk * cz
            pltpu.sync_copy(acc.at[pl.ds(r, cz), :],
                            out_hbm.at[ci, pl.ds(r, cz), :])

        @pl.when(si == NS - 1)
        def _flush_tail():
            pltpu.sync_copy(acc.at[pl.ds(tail_row, tail), :],
                            out_hbm.at[ci, pl.ds(tail_row, tail), :])

    return sc_edge


def kernel(x, edge_index, edge_attr, lin_W, lin_b, c1_Wf, c1_bf, c1_Ws, c1_bs,
           mu_Wf, mu_bf, mu_Ws, mu_bs, ls_Wf, ls_bf, ls_Ws, ls_bs):
    n, d = x.shape
    e = edge_index.shape[1]
    de = edge_attr.shape[1]
    src = edge_index[0]
    dst = edge_index[1]

    ew = -(-(e // NW) // B_EDGE) * B_EDGE
    while (ew // B_EDGE) % 2 != 0:
        ew += B_EDGE
    e_pad = NW * ew
    pad = e_pad - e
    dstp = jnp.concatenate([dst, jnp.zeros((pad,), jnp.int32)])
    srcp = jnp.concatenate([src, jnp.zeros((pad,), jnp.int32)])
    ea_pad = jnp.concatenate(
        [edge_attr, jnp.zeros((pad, de), jnp.float32)])
    eb = jnp.stack([dstp.reshape(e_pad // B_EDGE, B_EDGE),
                    srcp.reshape(e_pad // B_EDGE, B_EDGE)], axis=1)
    zeros = jnp.zeros((208, d), jnp.float32)

    def pack(wf, ws):
        w_dst = jnp.concatenate([wf[:d], ws[:d]], axis=1)
        w_src = jnp.concatenate([wf[d:2 * d], ws[d:2 * d]], axis=1)
        return w_dst, w_src, wf[2 * d:], ws[2 * d:]

    sc_edge = _make_sc_edge(n, d, e_pad)
    cbn = 8
    for cand in range(4224, 7, -8):
        if e_pad % cand == 0:
            cbn = cand
            break

    c1_wd, c1_ws_, c1_wfe, c1_wse = pack(c1_Wf, c1_Ws)
    mu_wd, mu_ws_, mu_wfe, mu_wse = pack(mu_Wf, mu_Ws)
    ls_wd, ls_ws_, ls_wfe, ls_wse = pack(ls_Wf, ls_Ws)

    h0 = _tc_mm_bias(x, lin_W, lin_b, bn=2000)

    pd, ps = _tc_proj2(h0, c1_wd, c1_ws_)
    ce = _tc_edgec(ea_pad, c1_wfe, c1_wse, c1_bf, c1_bs, e, bn=cbn)
    agg = sc_edge(eb, pd, ps, ce, zeros)

    h1, pd_mu, ps_mu, pd_ls, ps_ls = _tc_combine_proj4(
        h0, agg[0], agg[1], mu_wd, mu_ws_, ls_wd, ls_ws_)

    ce_mu = _tc_edgec(ea_pad, mu_wfe, mu_wse, mu_bf, mu_bs, e, bn=cbn)
    agg_mu = sc_edge(eb, pd_mu, ps_mu, ce_mu, zeros)
    mu = _tc_combine(h1, agg_mu[0], agg_mu[1])

    ce_ls = _tc_edgec(ea_pad, ls_wfe, ls_wse, ls_bf, ls_bs, e, bn=cbn)
    agg_ls = sc_edge(eb, pd_ls, ps_ls, ce_ls, zeros)
    logstd = _tc_combine(h1, agg_ls[0], agg_ls[1])

    return (mu, logstd)


# speedup vs baseline: 1.2323x; 1.2323x over previous
import functools

import jax
import jax.numpy as jnp
from jax import lax
from jax.experimental import pallas as pl
from jax.experimental.pallas import tpu as pltpu
from jax.experimental.pallas import tpu_sc as plsc

NC = 2
NS = 16
LANES = 16
NW = NC * NS

B_EDGE = 24


def _mm_bias_body(x_ref, w_ref, b_ref, o_ref):
    o_ref[...] = (
        jnp.dot(x_ref[...], w_ref[...], preferred_element_type=jnp.float32)
        + b_ref[...]
    )


def _tc_mm_bias(x, w, b, bn):
    n, d = x.shape
    k = w.shape[1]
    return pl.pallas_call(
        _mm_bias_body,
        grid=(n // bn,),
        in_specs=[
            pl.BlockSpec((bn, d), lambda i: (i, 0)),
            pl.BlockSpec((d, k), lambda i: (0, 0)),
            pl.BlockSpec((1, k), lambda i: (0, 0)),
        ],
        out_specs=pl.BlockSpec((bn, k), lambda i: (i, 0)),
        out_shape=jax.ShapeDtypeStruct((n, k), jnp.float32),
    )(x, w, b.reshape(1, k))


def _proj2_body(h_ref, w1_ref, w2_ref, o1_ref, o2_ref):
    h = h_ref[...]
    o1_ref[...] = jnp.dot(h, w1_ref[...], preferred_element_type=jnp.float32)
    o2_ref[...] = jnp.dot(h, w2_ref[...], preferred_element_type=jnp.float32)


def _tc_proj2(h, w1, w2, bn=2000):
    n, d = h.shape
    k = w1.shape[1]
    out = jax.ShapeDtypeStruct((n, k), jnp.float32)
    return pl.pallas_call(
        _proj2_body,
        grid=(n // bn,),
        in_specs=[
            pl.BlockSpec((bn, d), lambda i: (i, 0)),
            pl.BlockSpec((d, k), lambda i: (0, 0)),
            pl.BlockSpec((d, k), lambda i: (0, 0)),
        ],
        out_specs=[pl.BlockSpec((bn, k), lambda i: (i, 0))] * 2,
        out_shape=[out, out],
    )(h, w1, w2)


def _combine_proj4_body(h_ref, a0_ref, a1_ref, w1_ref, w2_ref, w3_ref, w4_ref,
                        hn_ref, o1_ref, o2_ref, o3_ref, o4_ref):
    hn = jnp.maximum(h_ref[...] + a0_ref[...] + a1_ref[...], 0.0)
    hn_ref[...] = hn
    for w_ref, o_ref in ((w1_ref, o1_ref), (w2_ref, o2_ref),
                         (w3_ref, o3_ref), (w4_ref, o4_ref)):
        o_ref[...] = jnp.dot(hn, w_ref[...], preferred_element_type=jnp.float32)


def _tc_combine_proj4(h, a0, a1, w1, w2, w3, w4, bn=1000):
    n, d = h.shape
    k = w1.shape[1]
    big = jax.ShapeDtypeStruct((n, k), jnp.float32)
    bspec = pl.BlockSpec((bn, d), lambda i: (i, 0))
    wspec = pl.BlockSpec((d, k), lambda i: (0, 0))
    ospec = pl.BlockSpec((bn, k), lambda i: (i, 0))
    return pl.pallas_call(
        _combine_proj4_body,
        grid=(n // bn,),
        in_specs=[bspec, bspec, bspec, wspec, wspec, wspec, wspec],
        out_specs=[bspec, ospec, ospec, ospec, ospec],
        out_shape=[jax.ShapeDtypeStruct((n, d), jnp.float32), big, big, big, big],
    )(h, a0, a1, w1, w2, w3, w4)


def _edgec_body(ea_ref, wf_ref, ws_ref, bf_ref, bs_ref, o_ref, *, bn, e_real):
    ea = ea_ref[...]
    cf = jnp.dot(ea, wf_ref[...], preferred_element_type=jnp.float32) + bf_ref[...]
    cs = jnp.dot(ea, ws_ref[...], preferred_element_type=jnp.float32) + bs_ref[...]
    rid = pl.program_id(0) * bn + lax.broadcasted_iota(jnp.int32, (bn, 1), 0)
    cs = jnp.where(rid >= e_real, -1e9, cs)
    o_ref[...] = jnp.concatenate([cf, cs], axis=1)


def _tc_edgec(ea_pad, wf_e, ws_e, bf, bs, e_real, bn=5008):
    n_pad, de = ea_pad.shape
    k = wf_e.shape[1]
    return pl.pallas_call(
        functools.partial(_edgec_body, bn=bn, e_real=e_real),
        grid=(n_pad // bn,),
        in_specs=[
            pl.BlockSpec((bn, de), lambda i: (i, 0)),
            pl.BlockSpec((de, k), lambda i: (0, 0)),
            pl.BlockSpec((de, k), lambda i: (0, 0)),
            pl.BlockSpec((1, k), lambda i: (0, 0)),
            pl.BlockSpec((1, k), lambda i: (0, 0)),
        ],
        out_specs=pl.BlockSpec((bn, 2 * k), lambda i: (i, 0)),
        out_shape=jax.ShapeDtypeStruct((n_pad, 2 * k), jnp.float32),
    )(ea_pad, wf_e, ws_e, bf.reshape(1, k), bs.reshape(1, k))


def _combine_body(h_ref, a0_ref, a1_ref, o_ref):
    o_ref[...] = h_ref[...] + a0_ref[...] + a1_ref[...]


def _tc_combine(h, a0, a1, bn=2000):
    n, d = h.shape
    spec = pl.BlockSpec((bn, d), lambda i: (i, 0))
    return pl.pallas_call(
        _combine_body,
        grid=(n // bn,),
        in_specs=[spec, spec, spec],
        out_specs=spec,
        out_shape=jax.ShapeDtypeStruct((n, d), jnp.float32),
    )(h, a0, a1)


def _gate_softplus16(gf, gs):
    u = jnp.exp(-jnp.abs(gs))
    t = u / (u + 2.0)
    t2 = t * t
    l1p = (2.0 * t) * (1.0 + t2 * (1.0 / 3.0 + t2 * (0.2 + t2 * (1.0 / 7.0))))
    sp = jnp.maximum(gs, 0.0) + l1p
    return sp / (1.0 + jnp.exp(-gf))


def _make_sc_edge(n, d, e):
    base_rows = (n // NS) // 8 * 8
    cz = 208
    ncz = base_rows // cz
    tail = n - NS * base_rows
    tail_row = NS * base_rows
    ew = e // NW
    nblk = ew // B_EDGE
    assert base_rows % cz == 0 and tail % 8 == 0
    assert e % NW == 0 and ew % B_EDGE == 0 and B_EDGE % 8 == 0
    assert nblk % 2 == 0 and nblk >= 4

    mesh = plsc.VectorSubcoreMesh(
        core_axis_name="c", subcore_axis_name="s", num_cores=NC,
        num_subcores=NS)

    @functools.partial(
        pl.kernel,
        mesh=mesh,
        out_type=jax.ShapeDtypeStruct((NC, n, d), jnp.float32),
        scratch_types=[
            pltpu.VMEM((2, B_EDGE), jnp.int32),
            pltpu.VMEM((2, B_EDGE), jnp.int32),
            pltpu.VMEM((B_EDGE,), jnp.int32),
            pltpu.VMEM((B_EDGE,), jnp.int32),
            pltpu.VMEM((B_EDGE, 2 * d), jnp.float32),
            pltpu.VMEM((B_EDGE, 2 * d), jnp.float32),
            pltpu.VMEM((B_EDGE, 2 * d), jnp.float32),
            pltpu.VMEM((B_EDGE, 2 * d), jnp.float32),
            pltpu.VMEM((B_EDGE, 2 * d), jnp.float32),
            pltpu.VMEM((B_EDGE, 2 * d), jnp.float32),
            pltpu.VMEM((B_EDGE, d), jnp.float32),
            pltpu.VMEM_SHARED((n, d), jnp.float32),
        ] + [pltpu.SemaphoreType.DMA] * 8,
    )
    def sc_edge(ei_hbm, pd_hbm, ps_hbm, c_hbm, zero_hbm, out_hbm,
                dsrc0, dsrc1, sdst0, sdst1, pdv0, pdv1, psv0, psv1, cv0, cv1,
                mv, acc, si0, si1, spd0, spd1, sps0, sps1, scc0, scc1):
        ci = lax.axis_index("c")
        si = lax.axis_index("s")
        wid = si * NC + ci
        base_blk = wid * nblk
        row0 = si * base_rows

        bufs = ((dsrc0, sdst0, pdv0, psv0, cv0, si0, spd0, sps0, scc0),
                (dsrc1, sdst1, pdv1, psv1, cv1, si1, spd1, sps1, scc1))

        for kk in range(ncz):
            pltpu.sync_copy(zero_hbm, acc.at[pl.ds(row0 + kk * cz, cz), :])

        @pl.when(si == NS - 1)
        def _zero_tail():
            pltpu.sync_copy(zero_hbm.at[pl.ds(0, tail), :],
                            acc.at[pl.ds(tail_row, tail), :])

        plsc.subcore_barrier()

        def idx_fetch(blk, b):
            buf = bufs[b]
            pltpu.async_copy(ei_hbm.at[blk], buf[0], buf[5])

        def idx_wait(b):
            buf = bufs[b]
            pltpu.make_async_copy(ei_hbm.at[0], buf[0], buf[5]).wait()

        def gather(blk, jb, b):
            dsrc = bufs[jb][0]
            _, _, pdv, psv, cv, _, spd, sps, scc = bufs[b]
            pltpu.async_copy(pd_hbm.at[dsrc.at[0]], pdv, spd)
            pltpu.async_copy(ps_hbm.at[dsrc.at[1]], psv, sps)
            pltpu.async_copy(c_hbm.at[pl.ds(blk * B_EDGE, B_EDGE), :], cv, scc)

        def consume(b, next_blk, do_fetch, unroll):
            dsrc, sdst, pdv, psv, cv, _, spd, sps, scc = bufs[b]
            pltpu.make_async_copy(pd_hbm.at[dsrc.at[0]], pdv, spd).wait()
            pltpu.make_async_copy(ps_hbm.at[dsrc.at[1]], psv, sps).wait()
            pltpu.make_async_copy(c_hbm.at[pl.ds(0, B_EDGE), :], cv, scc).wait()
            for q0 in range(0, B_EDGE - LANES + 1, LANES):
                sdst[pl.ds(q0, LANES)] = dsrc[0, pl.ds(q0, LANES)]
            if B_EDGE % LANES:
                q0 = B_EDGE - LANES
                sdst[pl.ds(q0, LANES)] = dsrc[0, pl.ds(q0, LANES)]
            if do_fetch:
                idx_fetch(next_blk, b)

            @plsc.parallel_loop(0, B_EDGE, 1, unroll=unroll)
            def edge_body(ei):
                for jj in range(d // LANES):
                    o = jj * LANES
                    cf = cv[ei, pl.ds(o, LANES)]
                    cs = cv[ei, pl.ds(d + o, LANES)]
                    gf = pdv[ei, pl.ds(o, LANES)] + psv[ei, pl.ds(o, LANES)] + cf
                    gs = (pdv[ei, pl.ds(d + o, LANES)]
                          + psv[ei, pl.ds(d + o, LANES)] + cs)
                    mv[ei, pl.ds(o, LANES)] = _gate_softplus16(gf, gs)
            pltpu.sync_copy(mv, acc.at[sdst], add=True)

        idx_fetch(base_blk, 0)
        idx_wait(0)
        gather(base_blk, 0, 0)
        idx_fetch(base_blk + 1, 1)

        def pair_body(i, carry):
            k = base_blk + 2 * i
            idx_wait(1)
            gather(k + 1, 1, 1)
            consume(0, k + 2, True, 2)
            idx_wait(0)
            gather(k + 2, 0, 0)
            consume(1, k + 3, True, 2)
            return carry

        lax.fori_loop(0, (nblk - 2) // 2, pair_body, 0, unroll=False)
        idx_wait(1)
        gather(base_blk + nblk - 1, 1, 1)
        consume(0, 0, False, 1)
        consume(1, 0, False, 1)
        plsc.subcore_barrier()

        for kk in range(ncz):
            r = row0 + kk * cz
            pltpu.sync_copy(acc.at[pl.ds(r, cz), :],
                            out_hbm.at[ci, pl.ds(r, cz), :])

        @pl.when(si == NS - 1)
        def _flush_tail():
            pltpu.sync_copy(acc.at[pl.ds(tail_row, tail), :],
                            out_hbm.at[ci, pl.ds(tail_row, tail), :])

    return sc_edge


def kernel(x, edge_index, edge_attr, lin_W, lin_b, c1_Wf, c1_bf, c1_Ws, c1_bs,
           mu_Wf, mu_bf, mu_Ws, mu_bs, ls_Wf, ls_bf, ls_Ws, ls_bs):
    n, d = x.shape
    e = edge_index.shape[1]
    de = edge_attr.shape[1]
    src = edge_index[0]
    dst = edge_index[1]

    ew = -(-(e // NW) // B_EDGE) * B_EDGE
    while (ew // B_EDGE) % 2 != 0:
        ew += B_EDGE
    e_pad = NW * ew
    pad = e_pad - e
    dstp = jnp.concatenate([dst, jnp.zeros((pad,), jnp.int32)])
    srcp = jnp.concatenate([src, jnp.zeros((pad,), jnp.int32)])
    ea_pad = jnp.concatenate(
        [edge_attr, jnp.zeros((pad, de), jnp.float32)])
    eb = jnp.stack([dstp.reshape(e_pad // B_EDGE, B_EDGE),
                    srcp.reshape(e_pad // B_EDGE, B_EDGE)], axis=1)
    zeros = jnp.zeros((208, d), jnp.float32)

    def pack(wf, ws):
        w_dst = jnp.concatenate([wf[:d], ws[:d]], axis=1)
        w_src = jnp.concatenate([wf[d:2 * d], ws[d:2 * d]], axis=1)
        return w_dst, w_src, wf[2 * d:], ws[2 * d:]

    sc_edge = _make_sc_edge(n, d, e_pad)
    cbn = 8
    for cand in range(4224, 7, -8):
        if e_pad % cand == 0:
            cbn = cand
            break

    c1_wd, c1_ws_, c1_wfe, c1_wse = pack(c1_Wf, c1_Ws)
    mu_wd, mu_ws_, mu_wfe, mu_wse = pack(mu_Wf, mu_Ws)
    ls_wd, ls_ws_, ls_wfe, ls_wse = pack(ls_Wf, ls_Ws)

    h0 = _tc_mm_bias(x, lin_W, lin_b, bn=2000)

    pd, ps = _tc_proj2(h0, c1_wd, c1_ws_)
    ce = _tc_edgec(ea_pad, c1_wfe, c1_wse, c1_bf, c1_bs, e, bn=cbn)
    agg = sc_edge(eb, pd, ps, ce, zeros)

    h1, pd_mu, ps_mu, pd_ls, ps_ls = _tc_combine_proj4(
        h0, agg[0], agg[1], mu_wd, mu_ws_, ls_wd, ls_ws_)

    ce_mu = _tc_edgec(ea_pad, mu_wfe, mu_wse, mu_bf, mu_bs, e, bn=cbn)
    agg_mu = sc_edge(eb, pd_mu, ps_mu, ce_mu, zeros)
    mu = _tc_combine(h1, agg_mu[0], agg_mu[1])

    ce_ls = _tc_edgec(ea_pad, ls_wfe, ls_wse, ls_bf, ls_bs, e, bn=cbn)
    agg_ls = sc_edge(eb, pd_ls, ps_ls, ce_ls, zeros)
    logstd = _tc_combine(h1, agg_ls[0], agg_ls[1])

    return (mu, logstd)


# speedup vs baseline: 1.2725x; 1.0326x over previous
import functools

import jax
import jax.numpy as jnp
from jax import lax
from jax.experimental import pallas as pl
from jax.experimental.pallas import tpu as pltpu
from jax.experimental.pallas import tpu_sc as plsc

NC = 2
NS = 16
LANES = 16
NW = NC * NS

B_EDGE = 24


def _mm_bias_body(x_ref, w_ref, b_ref, o_ref):
    o_ref[...] = (
        jnp.dot(x_ref[...], w_ref[...], preferred_element_type=jnp.float32)
        + b_ref[...]
    )


def _tc_mm_bias(x, w, b, bn):
    n, d = x.shape
    k = w.shape[1]
    return pl.pallas_call(
        _mm_bias_body,
        grid=(n // bn,),
        in_specs=[
            pl.BlockSpec((bn, d), lambda i: (i, 0)),
            pl.BlockSpec((d, k), lambda i: (0, 0)),
            pl.BlockSpec((1, k), lambda i: (0, 0)),
        ],
        out_specs=pl.BlockSpec((bn, k), lambda i: (i, 0)),
        out_shape=jax.ShapeDtypeStruct((n, k), jnp.float32),
    )(x, w, b.reshape(1, k))


def _proj2_body(h_ref, w1_ref, w2_ref, o1_ref, o2_ref):
    h = h_ref[...]
    o1_ref[...] = jnp.dot(h, w1_ref[...], preferred_element_type=jnp.float32)
    o2_ref[...] = jnp.dot(h, w2_ref[...], preferred_element_type=jnp.float32)


def _tc_proj2(h, w1, w2, bn=2000):
    n, d = h.shape
    k = w1.shape[1]
    out = jax.ShapeDtypeStruct((n, k), jnp.float32)
    return pl.pallas_call(
        _proj2_body,
        grid=(n // bn,),
        in_specs=[
            pl.BlockSpec((bn, d), lambda i: (i, 0)),
            pl.BlockSpec((d, k), lambda i: (0, 0)),
            pl.BlockSpec((d, k), lambda i: (0, 0)),
        ],
        out_specs=[pl.BlockSpec((bn, k), lambda i: (i, 0))] * 2,
        out_shape=[out, out],
    )(h, w1, w2)


def _combine_proj4_body(h_ref, a0_ref, a1_ref, w1_ref, w2_ref, w3_ref, w4_ref,
                        hn_ref, o1_ref, o2_ref, o3_ref, o4_ref):
    hn = jnp.maximum(h_ref[...] + a0_ref[...] + a1_ref[...], 0.0)
    hn_ref[...] = hn
    for w_ref, o_ref in ((w1_ref, o1_ref), (w2_ref, o2_ref),
                         (w3_ref, o3_ref), (w4_ref, o4_ref)):
        o_ref[...] = jnp.dot(hn, w_ref[...], preferred_element_type=jnp.float32)


def _tc_combine_proj4(h, a0, a1, w1, w2, w3, w4, bn=1000):
    n, d = h.shape
    k = w1.shape[1]
    big = jax.ShapeDtypeStruct((n, k), jnp.float32)
    bspec = pl.BlockSpec((bn, d), lambda i: (i, 0))
    wspec = pl.BlockSpec((d, k), lambda i: (0, 0))
    ospec = pl.BlockSpec((bn, k), lambda i: (i, 0))
    return pl.pallas_call(
        _combine_proj4_body,
        grid=(n // bn,),
        in_specs=[bspec, bspec, bspec, wspec, wspec, wspec, wspec],
        out_specs=[bspec, ospec, ospec, ospec, ospec],
        out_shape=[jax.ShapeDtypeStruct((n, d), jnp.float32), big, big, big, big],
    )(h, a0, a1, w1, w2, w3, w4)


def _edgec_body(ea_ref, wf_ref, ws_ref, bf_ref, bs_ref, o_ref, *, bn, e_real):
    ea = ea_ref[...]
    cf = jnp.dot(ea, wf_ref[...], preferred_element_type=jnp.float32) + bf_ref[...]
    cs = jnp.dot(ea, ws_ref[...], preferred_element_type=jnp.float32) + bs_ref[...]
    rid = pl.program_id(0) * bn + lax.broadcasted_iota(jnp.int32, (bn, 1), 0)
    cs = jnp.where(rid >= e_real, -1e9, cs)
    o_ref[...] = jnp.concatenate([cf, cs], axis=1)


def _tc_edgec(ea_pad, wf_e, ws_e, bf, bs, e_real, bn=5008):
    n_pad, de = ea_pad.shape
    k = wf_e.shape[1]
    return pl.pallas_call(
        functools.partial(_edgec_body, bn=bn, e_real=e_real),
        grid=(n_pad // bn,),
        in_specs=[
            pl.BlockSpec((bn, de), lambda i: (i, 0)),
            pl.BlockSpec((de, k), lambda i: (0, 0)),
            pl.BlockSpec((de, k), lambda i: (0, 0)),
            pl.BlockSpec((1, k), lambda i: (0, 0)),
            pl.BlockSpec((1, k), lambda i: (0, 0)),
        ],
        out_specs=pl.BlockSpec((bn, 2 * k), lambda i: (i, 0)),
        out_shape=jax.ShapeDtypeStruct((n_pad, 2 * k), jnp.float32),
    )(ea_pad, wf_e, ws_e, bf.reshape(1, k), bs.reshape(1, k))


def _combine_body(h_ref, a0_ref, a1_ref, o_ref):
    o_ref[...] = h_ref[...] + a0_ref[...] + a1_ref[...]


def _tc_combine(h, a0, a1, bn=2000):
    n, d = h.shape
    spec = pl.BlockSpec((bn, d), lambda i: (i, 0))
    return pl.pallas_call(
        _combine_body,
        grid=(n // bn,),
        in_specs=[spec, spec, spec],
        out_specs=spec,
        out_shape=jax.ShapeDtypeStruct((n, d), jnp.float32),
    )(h, a0, a1)


def _gate_softplus16(gf, gs):
    u = jnp.exp(-jnp.abs(gs))
    t = u / (u + 2.0)
    t2 = t * t
    l1p = (2.0 * t) * (1.0 + t2 * (1.0 / 3.0 + t2 * (0.2 + t2 * (1.0 / 7.0))))
    sp = jnp.maximum(gs, 0.0) + l1p
    return sp / (1.0 + jnp.exp(-gf))


def _make_sc_edge(n, d, e):
    base_rows = (n // NS) // 8 * 8
    cz = 208
    ncz = base_rows // cz
    tail = n - NS * base_rows
    tail_row = NS * base_rows
    ew = e // NW
    nblk = ew // B_EDGE
    assert base_rows % cz == 0 and tail % 8 == 0
    assert e % NW == 0 and ew % B_EDGE == 0 and B_EDGE % 8 == 0
    assert nblk % 2 == 0 and nblk >= 4

    mesh = plsc.VectorSubcoreMesh(
        core_axis_name="c", subcore_axis_name="s", num_cores=NC,
        num_subcores=NS)

    @functools.partial(
        pl.kernel,
        mesh=mesh,
        out_type=jax.ShapeDtypeStruct((NC, n, d), jnp.float32),
        scratch_types=[
            pltpu.VMEM((2, B_EDGE), jnp.int32),
            pltpu.VMEM((2, B_EDGE), jnp.int32),
            pltpu.VMEM((B_EDGE,), jnp.int32),
            pltpu.VMEM((B_EDGE,), jnp.int32),
            pltpu.VMEM((B_EDGE, 2 * d), jnp.float32),
            pltpu.VMEM((B_EDGE, 2 * d), jnp.float32),
            pltpu.VMEM((B_EDGE, 2 * d), jnp.float32),
            pltpu.VMEM((B_EDGE, 2 * d), jnp.float32),
            pltpu.VMEM((B_EDGE, 2 * d), jnp.float32),
            pltpu.VMEM((B_EDGE, 2 * d), jnp.float32),
            pltpu.VMEM((B_EDGE, d), jnp.float32),
            pltpu.VMEM_SHARED((n, d), jnp.float32),
        ] + [pltpu.SemaphoreType.DMA] * 8,
    )
    def sc_edge(ei_hbm, pd_hbm, ps_hbm, c_hbm, zero_hbm, out_hbm,
                dsrc0, dsrc1, sdst0, sdst1, pdv0, pdv1, psv0, psv1, cv0, cv1,
                mv, acc, si0, si1, spd0, spd1, sps0, sps1, scc0, scc1):
        ci = lax.axis_index("c")
        si = lax.axis_index("s")
        wid = si * NC + ci
        base_blk = wid * nblk
        row0 = si * base_rows

        bufs = ((dsrc0, sdst0, pdv0, psv0, cv0, si0, spd0, sps0, scc0),
                (dsrc1, sdst1, pdv1, psv1, cv1, si1, spd1, sps1, scc1))

        for kk in range(ncz):
            pltpu.sync_copy(zero_hbm, acc.at[pl.ds(row0 + kk * cz, cz), :])

        @pl.when(si == NS - 1)
        def _zero_tail():
            pltpu.sync_copy(zero_hbm.at[pl.ds(0, tail), :],
                            acc.at[pl.ds(tail_row, tail), :])

        plsc.subcore_barrier()

        def idx_fetch(blk, b):
            buf = bufs[b]
            pltpu.async_copy(ei_hbm.at[blk], buf[0], buf[5])

        def idx_wait(b):
            buf = bufs[b]
            pltpu.make_async_copy(ei_hbm.at[0], buf[0], buf[5]).wait()

        def gather(blk, jb, b):
            dsrc = bufs[jb][0]
            _, _, pdv, psv, cv, _, spd, sps, scc = bufs[b]
            pltpu.async_copy(pd_hbm.at[dsrc.at[0]], pdv, spd)
            pltpu.async_copy(ps_hbm.at[dsrc.at[1]], psv, sps)
            pltpu.async_copy(c_hbm.at[pl.ds(blk * B_EDGE, B_EDGE), :], cv, scc)

        def consume(b, next_blk, do_fetch, unroll):
            dsrc, sdst, pdv, psv, cv, _, spd, sps, scc = bufs[b]
            pltpu.make_async_copy(pd_hbm.at[dsrc.at[0]], pdv, spd).wait()
            pltpu.make_async_copy(ps_hbm.at[dsrc.at[1]], psv, sps).wait()
            pltpu.make_async_copy(c_hbm.at[pl.ds(0, B_EDGE), :], cv, scc).wait()
            for q0 in range(0, B_EDGE - LANES + 1, LANES):
                sdst[pl.ds(q0, LANES)] = dsrc[0, pl.ds(q0, LANES)]
            if B_EDGE % LANES:
                q0 = B_EDGE - LANES
                sdst[pl.ds(q0, LANES)] = dsrc[0, pl.ds(q0, LANES)]
            if do_fetch:
                idx_fetch(next_blk, b)

            @plsc.parallel_loop(0, B_EDGE, 1, unroll=unroll)
            def edge_body(ei):
                for jj in range(d // LANES):
                    o = jj * LANES
                    cf = cv[ei, pl.ds(o, LANES)]
                    cs = cv[ei, pl.ds(d + o, LANES)]
                    gf = pdv[ei, pl.ds(o, LANES)] + psv[ei, pl.ds(o, LANES)] + cf
                    gs = (pdv[ei, pl.ds(d + o, LANES)]
                          + psv[ei, pl.ds(d + o, LANES)] + cs)
                    mv[ei, pl.ds(o, LANES)] = _gate_softplus16(gf, gs)
            pltpu.sync_copy(mv, acc.at[sdst], add=True)

        idx_fetch(base_blk, 0)
        idx_wait(0)
        gather(base_blk, 0, 0)
        idx_fetch(base_blk + 1, 1)

        def pair_body(i, carry):
            k = base_blk + 2 * i
            idx_wait(1)
            gather(k + 1, 1, 1)
            consume(0, k + 2, True, 3)
            idx_wait(0)
            gather(k + 2, 0, 0)
            consume(1, k + 3, True, 3)
            return carry

        lax.fori_loop(0, (nblk - 2) // 2, pair_body, 0, unroll=False)
        idx_wait(1)
        gather(base_blk + nblk - 1, 1, 1)
        consume(0, 0, False, 1)
        consume(1, 0, False, 1)
        plsc.subcore_barrier()

        for kk in range(ncz):
            r = row0 + kk * cz
            pltpu.sync_copy(acc.at[pl.ds(r, cz), :],
                            out_hbm.at[ci, pl.ds(r, cz), :])

        @pl.when(si == NS - 1)
        def _flush_tail():
            pltpu.sync_copy(acc.at[pl.ds(tail_row, tail), :],
                            out_hbm.at[ci, pl.ds(tail_row, tail), :])

    return sc_edge


def kernel(x, edge_index, edge_attr, lin_W, lin_b, c1_Wf, c1_bf, c1_Ws, c1_bs,
           mu_Wf, mu_bf, mu_Ws, mu_bs, ls_Wf, ls_bf, ls_Ws, ls_bs):
    n, d = x.shape
    e = edge_index.shape[1]
    de = edge_attr.shape[1]
    src = edge_index[0]
    dst = edge_index[1]

    ew = -(-(e // NW) // B_EDGE) * B_EDGE
    while (ew // B_EDGE) % 2 != 0:
        ew += B_EDGE
    e_pad = NW * ew
    pad = e_pad - e
    dstp = jnp.concatenate([dst, jnp.zeros((pad,), jnp.int32)])
    srcp = jnp.concatenate([src, jnp.zeros((pad,), jnp.int32)])
    ea_pad = jnp.concatenate(
        [edge_attr, jnp.zeros((pad, de), jnp.float32)])
    eb = jnp.stack([dstp.reshape(e_pad // B_EDGE, B_EDGE),
                    srcp.reshape(e_pad // B_EDGE, B_EDGE)], axis=1)
    zeros = jnp.zeros((208, d), jnp.float32)

    def pack(wf, ws):
        w_dst = jnp.concatenate([wf[:d], ws[:d]], axis=1)
        w_src = jnp.concatenate([wf[d:2 * d], ws[d:2 * d]], axis=1)
        return w_dst, w_src, wf[2 * d:], ws[2 * d:]

    sc_edge = _make_sc_edge(n, d, e_pad)
    cbn = 8
    for cand in range(4224, 7, -8):
        if e_pad % cand == 0:
            cbn = cand
            break

    c1_wd, c1_ws_, c1_wfe, c1_wse = pack(c1_Wf, c1_Ws)
    mu_wd, mu_ws_, mu_wfe, mu_wse = pack(mu_Wf, mu_Ws)
    ls_wd, ls_ws_, ls_wfe, ls_wse = pack(ls_Wf, ls_Ws)

    h0 = _tc_mm_bias(x, lin_W, lin_b, bn=2000)

    pd, ps = _tc_proj2(h0, c1_wd, c1_ws_)
    ce = _tc_edgec(ea_pad, c1_wfe, c1_wse, c1_bf, c1_bs, e, bn=cbn)
    agg = sc_edge(eb, pd, ps, ce, zeros)

    h1, pd_mu, ps_mu, pd_ls, ps_ls = _tc_combine_proj4(
        h0, agg[0], agg[1], mu_wd, mu_ws_, ls_wd, ls_ws_)

    ce_mu = _tc_edgec(ea_pad, mu_wfe, mu_wse, mu_bf, mu_bs, e, bn=cbn)
    agg_mu = sc_edge(eb, pd_mu, ps_mu, ce_mu, zeros)
    mu = _tc_combine(h1, agg_mu[0], agg_mu[1])

    ce_ls = _tc_edgec(ea_pad, ls_wfe, ls_wse, ls_bf, ls_bs, e, bn=cbn)
    agg_ls = sc_edge(eb, pd_ls, ps_ls, ce_ls, zeros)
    logstd = _tc_combine(h1, agg_ls[0], agg_ls[1])

    return (mu, logstd)
